# baseline (speedup 1.0000x reference)
import functools

import jax
import jax.numpy as jnp
from jax import lax
from jax.experimental import pallas as pl
from jax.experimental.pallas import tpu as pltpu
from jax.experimental.pallas import tpu_sc as plsc

NC = 2
NS = 16
NW = NC * NS
LANES = 16
CHUNK = 112
RB = 3
IB = 6


def _make_sc_scatter(n_pad, d, k0, k1):
    assert k0 % IB == 0 and k1 % IB == 0 and IB % RB == 0
    rows_per_sub = n_pad // NS

    mesh = plsc.VectorSubcoreMesh(core_axis_name="c", subcore_axis_name="s")

    @functools.partial(
        pl.kernel,
        mesh=mesh,
        out_type=jax.ShapeDtypeStruct((NC, n_pad, d), jnp.float32),
        scratch_types=[
            pltpu.VMEM((IB, CHUNK), jnp.int32),
            pltpu.VMEM((IB, CHUNK), jnp.int32),
            pltpu.VMEM((IB, CHUNK), jnp.float32),
            pltpu.VMEM((RB, CHUNK, d), jnp.float32),
            pltpu.VMEM_SHARED((n_pad, d), jnp.float32),
            pltpu.SemaphoreType.DMA((RB,)),
            pltpu.SemaphoreType.DMA((IB,)),
            pltpu.SemaphoreType.DMA((2,)),
        ],
    )
    def sc_scatter(m_hbm, src_hbm, dst_hbm, w_hbm, out_hbm,
                   idx_s, idx_d, w_v, rows, agg_sp, gsem, isem, ssem):
        c = lax.axis_index("c")
        s = lax.axis_index("s")
        off = c * k0
        n_my = jnp.where(c == 0, k0, k1)

        def fetch_idx(k, u):
            pltpu.async_copy(src_hbm.at[s, off + k], idx_s.at[u], isem.at[u])
            pltpu.async_copy(dst_hbm.at[s, off + k], idx_d.at[u], isem.at[u])
            pltpu.async_copy(w_hbm.at[s, off + k], w_v.at[u], isem.at[u])

        def wait_idx(k, u):
            pltpu.make_async_copy(src_hbm.at[s, off + k], idx_s.at[u],
                                  isem.at[u]).wait()
            pltpu.make_async_copy(dst_hbm.at[s, off + k], idx_d.at[u],
                                  isem.at[u]).wait()
            pltpu.make_async_copy(w_hbm.at[s, off + k], w_v.at[u],
                                  isem.at[u]).wait()

        half = CHUNK // 2

        def gather(k, u, b):
            pltpu.async_copy(m_hbm.at[idx_s.at[u, pl.ds(0, half)]],
                             rows.at[b, pl.ds(0, half)], gsem.at[b])
            pltpu.async_copy(m_hbm.at[idx_s.at[u, pl.ds(half, half)]],
                             rows.at[b, pl.ds(half, half)], gsem.at[b])

        def wait_gather(u, b):
            pltpu.make_async_copy(m_hbm.at[idx_s.at[u, pl.ds(0, half)]],
                                  rows.at[b, pl.ds(0, half)],
                                  gsem.at[b]).wait()
            pltpu.make_async_copy(m_hbm.at[idx_s.at[u, pl.ds(half, half)]],
                                  rows.at[b, pl.ds(half, half)],
                                  gsem.at[b]).wait()

        sl_my = pl.ds(s * rows_per_sub, rows_per_sub)
        zvec = jnp.zeros((LANES,), jnp.float32)

        def zrow_body(i, carry):
            for j in range(d // LANES):
                rows[0, i, pl.ds(j * LANES, LANES)] = zvec
            return carry

        lax.fori_loop(0, CHUNK, zrow_body, 0)
        n_full = rows_per_sub // CHUNK
        rem = rows_per_sub % CHUNK
        for t in range(n_full):
            pltpu.sync_copy(
                rows.at[0],
                agg_sp.at[pl.ds(s * rows_per_sub + t * CHUNK, CHUNK)])
        if rem:
            pltpu.sync_copy(
                rows.at[0, pl.ds(0, rem)],
                agg_sp.at[pl.ds(s * rows_per_sub + n_full * CHUNK, rem)])
        plsc.subcore_barrier()

        for j in range(IB - 1):
            fetch_idx(j, j)
        wait_idx(0, 0)
        gather(0, 0, 0)
        wait_idx(1, 1)
        gather(1, 1, 1)

        def block_body(kk, carry):
            for z in range(IB):
                k = kk * IB + z
                b = z % RB
                u = z
                wait_gather(u, b)

                def group_body(g, carry2):
                    wv = w_v[u, pl.ds(g * LANES, LANES)]
                    for lane in range(LANES):
                        wi = wv[lane]
                        i = g * LANES + lane
                        for j in range(d // LANES):
                            sl = pl.ds(j * LANES, LANES)
                            rows[b, i, sl] = rows[b, i, sl] * wi
                    return carry2

                lax.fori_loop(0, CHUNK // LANES, group_body, 0)

                @pl.when(k > 0)
                def _():
                    pltpu.make_async_copy(
                        rows.at[(z + 2) % RB],
                        agg_sp.at[idx_d.at[(z + IB - 1) % IB]],
                        ssem.at[(z + 1) % 2]).wait()

                @pl.when(k + IB - 1 < n_my)
                def _():
                    fetch_idx(k + IB - 1, (z + IB - 1) % IB)

                @pl.when(k + 2 < n_my)
                def _():
                    wait_idx(k + 2, (z + 2) % IB)
                    gather(k + 2, (z + 2) % IB, (z + 2) % RB)

                pltpu.async_copy(rows.at[b], agg_sp.at[idx_d.at[u]],
                                 ssem.at[z % 2], add=True)
            return carry

        lax.fori_loop(0, n_my // IB, block_body, 0)
        pltpu.make_async_copy(rows.at[0], agg_sp.at[idx_d.at[0]],
                              ssem.at[1]).wait()
        plsc.subcore_barrier()
        pltpu.sync_copy(agg_sp.at[sl_my], out_hbm.at[c, sl_my])

    return sc_scatter


ROW_BLK = 2000


def _tc_pre_body(h_ref, wm_ref, whh_ref, bhh_ref, m_ref, gh_ref):
    h = h_ref[...]
    m_ref[...] = jnp.dot(h, wm_ref[...], preferred_element_type=jnp.float32)
    gh_ref[...] = (jnp.dot(h, whh_ref[...], preferred_element_type=jnp.float32)
                   + bhh_ref[...])


def _tc_pre(h, wm, whh_t, bhh, n, d):
    grid = (n // ROW_BLK,)
    return pl.pallas_call(
        _tc_pre_body,
        grid=grid,
        in_specs=[
            pl.BlockSpec((ROW_BLK, d), lambda i: (i, 0)),
            pl.BlockSpec((d, d), lambda i: (0, 0)),
            pl.BlockSpec((d, 3 * d), lambda i: (0, 0)),
            pl.BlockSpec((1, 3 * d), lambda i: (0, 0)),
        ],
        out_specs=[
            pl.BlockSpec((ROW_BLK, d), lambda i: (i, 0)),
            pl.BlockSpec((ROW_BLK, 3 * d), lambda i: (i, 0)),
        ],
        out_shape=[
            jax.ShapeDtypeStruct((n, d), jnp.float32),
            jax.ShapeDtypeStruct((n, 3 * d), jnp.float32),
        ],
    )(h, wm, whh_t, bhh)


def _gru_update(agg, h, gh, wih_t, bih, d):
    gi = jnp.dot(agg, wih_t, preferred_element_type=jnp.float32) + bih
    r = jax.nn.sigmoid(gi[:, :d] + gh[:, :d])
    z = jax.nn.sigmoid(gi[:, d:2 * d] + gh[:, d:2 * d])
    nn = jnp.tanh(gi[:, 2 * d:] + r * gh[:, 2 * d:])
    return (1.0 - z) * nn + z * h


def _tc_gru_mid_body(d, aggp_ref, h_ref, gh_ref, wih_ref, bih_ref,
                     wm_ref, whh_ref, bhh_ref,
                     h_out_ref, m_out_ref, gh_out_ref):
    agg = aggp_ref[0] + aggp_ref[1]
    h_new = _gru_update(agg, h_ref[...], gh_ref[...], wih_ref[...],
                        bih_ref[...], d)
    h_out_ref[...] = h_new
    m_out_ref[...] = jnp.dot(h_new, wm_ref[...],
                             preferred_element_type=jnp.float32)
    gh_out_ref[...] = (jnp.dot(h_new, whh_ref[...],
                               preferred_element_type=jnp.float32)
                       + bhh_ref[...])


def _tc_gru_last_body(d, aggp_ref, h_ref, gh_ref, wih_ref, bih_ref,
                      h_out_ref):
    agg = aggp_ref[0] + aggp_ref[1]
    h_out_ref[...] = _gru_update(agg, h_ref[...], gh_ref[...], wih_ref[...],
                                 bih_ref[...], d)


def _tc_gru(aggp, h, gh, wih_t, bih, wm_next, whh_t, bhh, n, d, last):
    grid = (n // ROW_BLK,)
    in_specs = [
        pl.BlockSpec((NC, ROW_BLK, d), lambda i: (0, i, 0)),
        pl.BlockSpec((ROW_BLK, d), lambda i: (i, 0)),
        pl.BlockSpec((ROW_BLK, 3 * d), lambda i: (i, 0)),
        pl.BlockSpec((d, 3 * d), lambda i: (0, 0)),
        pl.BlockSpec((1, 3 * d), lambda i: (0, 0)),
    ]
    if last:
        return pl.pallas_call(
            functools.partial(_tc_gru_last_body, d),
            grid=grid,
            in_specs=in_specs,
            out_specs=pl.BlockSpec((ROW_BLK, d), lambda i: (i, 0)),
            out_shape=jax.ShapeDtypeStruct((n, d), jnp.float32),
        )(aggp, h, gh, wih_t, bih)
    in_specs += [
        pl.BlockSpec((d, d), lambda i: (0, 0)),
        pl.BlockSpec((d, 3 * d), lambda i: (0, 0)),
        pl.BlockSpec((1, 3 * d), lambda i: (0, 0)),
    ]
    return pl.pallas_call(
        functools.partial(_tc_gru_mid_body, d),
        grid=grid,
        in_specs=in_specs,
        out_specs=[
            pl.BlockSpec((ROW_BLK, d), lambda i: (i, 0)),
            pl.BlockSpec((ROW_BLK, d), lambda i: (i, 0)),
            pl.BlockSpec((ROW_BLK, 3 * d), lambda i: (i, 0)),
        ],
        out_shape=[
            jax.ShapeDtypeStruct((n, d), jnp.float32),
            jax.ShapeDtypeStruct((n, d), jnp.float32),
            jax.ShapeDtypeStruct((n, 3 * d), jnp.float32),
        ],
    )(aggp, h, gh, wih_t, bih, wm_next, whh_t, bhh)


def kernel(x, edge_index, edge_attr, weight, w_ih, w_hh, b_ih, b_hh):
    n, d = x.shape
    num_layers = weight.shape[0]
    e = edge_attr.shape[0]

    kt = -(-e // (NS * CHUNK * 3 * IB)) * 3 * IB
    k0 = (kt * 23 // 30 // IB) * IB
    k1 = kt - k0
    e_pad = NS * kt * CHUNK
    src = edge_index[0]
    dst = edge_index[1]
    if e_pad != e:
        pad = e_pad - e
        src = jnp.concatenate([src, jnp.zeros((pad,), jnp.int32)])
        dst = jnp.concatenate([dst, jnp.zeros((pad,), jnp.int32)])
        edge_attr = jnp.concatenate([edge_attr, jnp.zeros((pad,), jnp.float32)])
    src = src.reshape(NS, kt, CHUNK)
    dst = dst.reshape(NS, kt, CHUNK)
    edge_attr = edge_attr.reshape(NS, kt, CHUNK)

    whh_t = w_hh.T
    wih_t = w_ih.T
    bhh = b_hh.reshape(1, 3 * d)
    bih = b_ih.reshape(1, 3 * d)
    n_pad = -(-n // (8 * NS)) * (8 * NS)

    sc_scatter = _make_sc_scatter(n_pad, d, k0, k1)

    h = x
    m, gh = _tc_pre(h, weight[0], whh_t, bhh, n, d)
    for i in range(num_layers):
        aggp = sc_scatter(m, src, dst, edge_attr)
        last = i == num_layers - 1
        wm_next = weight[i + 1] if not last else weight[0]
        if last:
            h = _tc_gru(aggp, h, gh, wih_t, bih, wm_next, whh_t, bhh,
                        n, d, last=True)
        else:
            h, m, gh = _tc_gru(aggp, h, gh, wih_t, bih, wm_next, whh_t, bhh,
                               n, d, last=False)
    return h


# speedup vs baseline: 1.0197x; 1.0197x over previous
import functools

import jax
import jax.numpy as jnp
from jax import lax
from jax.experimental import pallas as pl
from jax.experimental.pallas import tpu as pltpu
from jax.experimental.pallas import tpu_sc as plsc

NC = 2
NS = 16
NW = NC * NS
LANES = 16
CHUNK = 112
RB = 3
IB = 6


def _make_sc_scatter(n_pad, d, k0, k1):
    assert k0 % IB == 0 and k1 % IB == 0 and IB % RB == 0
    rows_per_sub = n_pad // NS

    mesh = plsc.VectorSubcoreMesh(core_axis_name="c", subcore_axis_name="s")

    @functools.partial(
        pl.kernel,
        mesh=mesh,
        out_type=jax.ShapeDtypeStruct((NC, n_pad, d), jnp.float32),
        scratch_types=[
            pltpu.VMEM((IB, CHUNK), jnp.int32),
            pltpu.VMEM((IB, CHUNK), jnp.int32),
            pltpu.VMEM((IB, CHUNK), jnp.float32),
            pltpu.VMEM((RB, CHUNK, d), jnp.float32),
            pltpu.VMEM_SHARED((n_pad, d), jnp.float32),
            pltpu.SemaphoreType.DMA((RB,)),
            pltpu.SemaphoreType.DMA((IB,)),
            pltpu.SemaphoreType.DMA((2,)),
        ],
    )
    def sc_scatter(m_hbm, src_hbm, dst_hbm, w_hbm, out_hbm,
                   idx_s, idx_d, w_v, rows, agg_sp, gsem, isem, ssem):
        c = lax.axis_index("c")
        s = lax.axis_index("s")
        off = c * k0
        n_my = jnp.where(c == 0, k0, k1)

        def fetch_idx(k, u):
            pltpu.async_copy(src_hbm.at[s, off + k], idx_s.at[u], isem.at[u])
            pltpu.async_copy(dst_hbm.at[s, off + k], idx_d.at[u], isem.at[u])
            pltpu.async_copy(w_hbm.at[s, off + k], w_v.at[u], isem.at[u])

        def wait_idx(k, u):
            pltpu.make_async_copy(src_hbm.at[s, off + k], idx_s.at[u],
                                  isem.at[u]).wait()
            pltpu.make_async_copy(dst_hbm.at[s, off + k], idx_d.at[u],
                                  isem.at[u]).wait()
            pltpu.make_async_copy(w_hbm.at[s, off + k], w_v.at[u],
                                  isem.at[u]).wait()

        def gather(k, u, b):
            pltpu.async_copy(m_hbm.at[idx_s.at[u]], rows.at[b], gsem.at[b])

        def wait_gather(u, b):
            pltpu.make_async_copy(m_hbm.at[idx_s.at[u]], rows.at[b],
                                  gsem.at[b]).wait()

        sl_my = pl.ds(s * rows_per_sub, rows_per_sub)
        zvec = jnp.zeros((LANES,), jnp.float32)

        def zrow_body(i, carry):
            for j in range(d // LANES):
                rows[0, i, pl.ds(j * LANES, LANES)] = zvec
            return carry

        lax.fori_loop(0, CHUNK, zrow_body, 0)
        n_full = rows_per_sub // CHUNK
        rem = rows_per_sub % CHUNK
        for t in range(n_full):
            pltpu.sync_copy(
                rows.at[0],
                agg_sp.at[pl.ds(s * rows_per_sub + t * CHUNK, CHUNK)])
        if rem:
            pltpu.sync_copy(
                rows.at[0, pl.ds(0, rem)],
                agg_sp.at[pl.ds(s * rows_per_sub + n_full * CHUNK, rem)])
        plsc.subcore_barrier()

        for j in range(IB - 1):
            fetch_idx(j, j)
        wait_idx(0, 0)
        gather(0, 0, 0)
        wait_idx(1, 1)
        gather(1, 1, 1)

        def block_body(kk, carry):
            for z in range(IB):
                k = kk * IB + z
                b = z % RB
                u = z
                wait_gather(u, b)

                def group_body(g, carry2):
                    wv = w_v[u, pl.ds(g * LANES, LANES)]
                    for lane in range(LANES):
                        wi = wv[lane]
                        i = g * LANES + lane
                        for j in range(d // LANES):
                            sl = pl.ds(j * LANES, LANES)
                            rows[b, i, sl] = rows[b, i, sl] * wi
                    return carry2

                lax.fori_loop(0, CHUNK // LANES, group_body, 0)

                @pl.when(k > 0)
                def _():
                    pltpu.make_async_copy(
                        rows.at[(z + 2) % RB],
                        agg_sp.at[idx_d.at[(z + IB - 1) % IB]],
                        ssem.at[(z + 1) % 2]).wait()

                @pl.when(k + IB - 1 < n_my)
                def _():
                    fetch_idx(k + IB - 1, (z + IB - 1) % IB)

                @pl.when(k + 2 < n_my)
                def _():
                    wait_idx(k + 2, (z + 2) % IB)
                    gather(k + 2, (z + 2) % IB, (z + 2) % RB)

                pltpu.async_copy(rows.at[b], agg_sp.at[idx_d.at[u]],
                                 ssem.at[z % 2], add=True)
            return carry

        lax.fori_loop(0, n_my // IB, block_body, 0)
        pltpu.make_async_copy(rows.at[0], agg_sp.at[idx_d.at[0]],
                              ssem.at[1]).wait()
        plsc.subcore_barrier()
        pltpu.sync_copy(agg_sp.at[sl_my], out_hbm.at[c, sl_my])

    return sc_scatter


ROW_BLK = 2000


def _tc_pre_body(h_ref, wm_ref, whh_ref, bhh_ref, m_ref, gh_ref):
    h = h_ref[...]
    m_ref[...] = jnp.dot(h, wm_ref[...], preferred_element_type=jnp.float32)
    gh_ref[...] = (jnp.dot(h, whh_ref[...], preferred_element_type=jnp.float32)
                   + bhh_ref[...])


def _tc_pre(h, wm, whh_t, bhh, n, d):
    grid = (n // ROW_BLK,)
    return pl.pallas_call(
        _tc_pre_body,
        grid=grid,
        in_specs=[
            pl.BlockSpec((ROW_BLK, d), lambda i: (i, 0)),
            pl.BlockSpec((d, d), lambda i: (0, 0)),
            pl.BlockSpec((d, 3 * d), lambda i: (0, 0)),
            pl.BlockSpec((1, 3 * d), lambda i: (0, 0)),
        ],
        out_specs=[
            pl.BlockSpec((ROW_BLK, d), lambda i: (i, 0)),
            pl.BlockSpec((ROW_BLK, 3 * d), lambda i: (i, 0)),
        ],
        out_shape=[
            jax.ShapeDtypeStruct((n, d), jnp.float32),
            jax.ShapeDtypeStruct((n, 3 * d), jnp.float32),
        ],
    )(h, wm, whh_t, bhh)


def _gru_update(agg, h, gh, wih_t, bih, d):
    gi = jnp.dot(agg, wih_t, preferred_element_type=jnp.float32) + bih
    r = jax.nn.sigmoid(gi[:, :d] + gh[:, :d])
    z = jax.nn.sigmoid(gi[:, d:2 * d] + gh[:, d:2 * d])
    nn = jnp.tanh(gi[:, 2 * d:] + r * gh[:, 2 * d:])
    return (1.0 - z) * nn + z * h


def _tc_gru_mid_body(d, aggp_ref, h_ref, gh_ref, wih_ref, bih_ref,
                     wm_ref, whh_ref, bhh_ref,
                     h_out_ref, m_out_ref, gh_out_ref):
    agg = aggp_ref[0] + aggp_ref[1]
    h_new = _gru_update(agg, h_ref[...], gh_ref[...], wih_ref[...],
                        bih_ref[...], d)
    h_out_ref[...] = h_new
    m_out_ref[...] = jnp.dot(h_new, wm_ref[...],
                             preferred_element_type=jnp.float32)
    gh_out_ref[...] = (jnp.dot(h_new, whh_ref[...],
                               preferred_element_type=jnp.float32)
                       + bhh_ref[...])


def _tc_gru_last_body(d, aggp_ref, h_ref, gh_ref, wih_ref, bih_ref,
                      h_out_ref):
    agg = aggp_ref[0] + aggp_ref[1]
    h_out_ref[...] = _gru_update(agg, h_ref[...], gh_ref[...], wih_ref[...],
                                 bih_ref[...], d)


def _tc_gru(aggp, h, gh, wih_t, bih, wm_next, whh_t, bhh, n, d, last):
    grid = (n // ROW_BLK,)
    in_specs = [
        pl.BlockSpec((NC, ROW_BLK, d), lambda i: (0, i, 0)),
        pl.BlockSpec((ROW_BLK, d), lambda i: (i, 0)),
        pl.BlockSpec((ROW_BLK, 3 * d), lambda i: (i, 0)),
        pl.BlockSpec((d, 3 * d), lambda i: (0, 0)),
        pl.BlockSpec((1, 3 * d), lambda i: (0, 0)),
    ]
    if last:
        return pl.pallas_call(
            functools.partial(_tc_gru_last_body, d),
            grid=grid,
            in_specs=in_specs,
            out_specs=pl.BlockSpec((ROW_BLK, d), lambda i: (i, 0)),
            out_shape=jax.ShapeDtypeStruct((n, d), jnp.float32),
        )(aggp, h, gh, wih_t, bih)
    in_specs += [
        pl.BlockSpec((d, d), lambda i: (0, 0)),
        pl.BlockSpec((d, 3 * d), lambda i: (0, 0)),
        pl.BlockSpec((1, 3 * d), lambda i: (0, 0)),
    ]
    return pl.pallas_call(
        functools.partial(_tc_gru_mid_body, d),
        grid=grid,
        in_specs=in_specs,
        out_specs=[
            pl.BlockSpec((ROW_BLK, d), lambda i: (i, 0)),
            pl.BlockSpec((ROW_BLK, d), lambda i: (i, 0)),
            pl.BlockSpec((ROW_BLK, 3 * d), lambda i: (i, 0)),
        ],
        out_shape=[
            jax.ShapeDtypeStruct((n, d), jnp.float32),
            jax.ShapeDtypeStruct((n, d), jnp.float32),
            jax.ShapeDtypeStruct((n, 3 * d), jnp.float32),
        ],
    )(aggp, h, gh, wih_t, bih, wm_next, whh_t, bhh)


def kernel(x, edge_index, edge_attr, weight, w_ih, w_hh, b_ih, b_hh):
    n, d = x.shape
    num_layers = weight.shape[0]
    e = edge_attr.shape[0]

    kt = -(-e // (NS * CHUNK * 3 * IB)) * 3 * IB
    k0 = (kt * 23 // 30 // IB) * IB
    k1 = kt - k0
    e_pad = NS * kt * CHUNK
    src = edge_index[0]
    dst = edge_index[1]
    if e_pad != e:
        pad = e_pad - e
        src = jnp.concatenate([src, jnp.zeros((pad,), jnp.int32)])
        dst = jnp.concatenate([dst, jnp.zeros((pad,), jnp.int32)])
        edge_attr = jnp.concatenate([edge_attr, jnp.zeros((pad,), jnp.float32)])
    src = src.reshape(NS, kt, CHUNK)
    dst = dst.reshape(NS, kt, CHUNK)
    edge_attr = edge_attr.reshape(NS, kt, CHUNK)

    whh_t = w_hh.T
    wih_t = w_ih.T
    bhh = b_hh.reshape(1, 3 * d)
    bih = b_ih.reshape(1, 3 * d)
    n_pad = -(-n // (8 * NS)) * (8 * NS)

    sc_scatter = _make_sc_scatter(n_pad, d, k0, k1)

    h = x
    m, gh = _tc_pre(h, weight[0], whh_t, bhh, n, d)
    for i in range(num_layers):
        aggp = sc_scatter(m, src, dst, edge_attr)
        last = i == num_layers - 1
        wm_next = weight[i + 1] if not last else weight[0]
        if last:
            h = _tc_gru(aggp, h, gh, wih_t, bih, wm_next, whh_t, bhh,
                        n, d, last=True)
        else:
            h, m, gh = _tc_gru(aggp, h, gh, wih_t, bih, wm_next, whh_t, bhh,
                               n, d, last=False)
    return h
